# Initial kernel scaffold; baseline (speedup 1.0000x reference)
import jax
import jax.numpy as jnp
from jax.experimental import pallas as pl


def kernel(data, table, sent_maxlen):
    raise NotImplementedError("write your pallas kernel here")


# speedup vs baseline: 12.4372x; 12.4372x over previous
import functools

import jax
import jax.numpy as jnp
from jax import lax
from jax.experimental import pallas as pl
from jax.experimental.pallas import tpu as pltpu
from jax.experimental.pallas import tpu_sc as plsc

_NC = 2
_NS = 16
_L = 16


def _sc_gather_sum(data_flat, table, B, W):
    S = W - 2
    V, D = table.shape
    NW = _NC * _NS
    BPW = B // NW
    G = 4
    GW = G * W
    NG = BPW // G
    NCH = (GW + 127) // 128

    mesh = plsc.VectorSubcoreMesh(core_axis_name="c", subcore_axis_name="s")

    @functools.partial(
        pl.kernel,
        out_type=jax.ShapeDtypeStruct((B, D), jnp.float32),
        mesh=mesh,
        scratch_types=[
            pltpu.VMEM((GW,), jnp.int32),
            pltpu.VMEM((GW, D), jnp.float32),
            pltpu.VMEM((BPW, D), jnp.float32),
            pltpu.SemaphoreType.DMA,
        ],
        compiler_params=pltpu.CompilerParams(use_tc_tiling_on_sc=False),
    )
    def sc_kernel(data_hbm, table_hbm, out_hbm, dbuf, rows, outbuf, sem):
        wid = lax.axis_index("s") * _NC + lax.axis_index("c")
        base = wid * BPW

        def group_step(g, _):
            off = (base + g * G) * W
            pltpu.sync_copy(data_hbm.at[pl.ds(off, GW)], dbuf)
            copies = []
            for c in range(NCH):
                cs = c * 128
                cl = min(128, GW - cs)
                copies.append(pltpu.async_copy(
                    table_hbm.at[dbuf.at[pl.ds(cs, cl)]],
                    rows.at[pl.ds(cs, cl)], sem))
            for cp in copies:
                cp.wait()

            for j in range(G):
                jo = j * W

                def acc_step(k, carry, jo=jo):
                    a0, b0, a1, b1, a2, b2, a3, b3 = carry
                    t = jo + k * 4
                    a0 = a0 + rows[t, pl.ds(0, _L)]
                    b0 = b0 + rows[t, pl.ds(_L, _L)]
                    a1 = a1 + rows[t + 1, pl.ds(0, _L)]
                    b1 = b1 + rows[t + 1, pl.ds(_L, _L)]
                    a2 = a2 + rows[t + 2, pl.ds(0, _L)]
                    b2 = b2 + rows[t + 2, pl.ds(_L, _L)]
                    a3 = a3 + rows[t + 3, pl.ds(0, _L)]
                    b3 = b3 + rows[t + 3, pl.ds(_L, _L)]
                    return a0, b0, a1, b1, a2, b2, a3, b3

                z = jnp.zeros((_L,), jnp.float32)
                a0, b0, a1, b1, a2, b2, a3, b3 = lax.fori_loop(
                    0, S // 4, acc_step, (z, z, z, z, z, z, z, z))
                outbuf[g * G + j, pl.ds(0, _L)] = (a0 + a1) + (a2 + a3)
                outbuf[g * G + j, pl.ds(_L, _L)] = (b0 + b1) + (b2 + b3)
            return 0

        lax.fori_loop(0, NG, group_step, 0)
        pltpu.sync_copy(outbuf, out_hbm.at[pl.ds(base, BPW)])

    return sc_kernel(data_flat, table)


def _tc_head(sums, data, t1):
    B, D = sums.shape
    S = data.shape[1] - 2

    def head(sums_ref, data_ref, t1_ref, loss_ref, acc_ref):
        tokens = data_ref[:, :S]
        cnt = jnp.sum((tokens == 1).astype(jnp.float32), axis=1, keepdims=True)
        lens = data_ref[:, S:S + 1].astype(jnp.float32)
        y = data_ref[:, S + 1:S + 2]
        pooled = (sums_ref[...] - cnt * t1_ref[...]) / lens
        col = lax.broadcasted_iota(jnp.int32, (B, D), 1)
        m = jnp.max(pooled, axis=1, keepdims=True)
        se = jnp.sum(jnp.exp(pooled - m), axis=1, keepdims=True)
        log_z = m + jnp.log(se)
        p_y = jnp.sum(jnp.where(col == y, pooled, 0.0), axis=1, keepdims=True)
        loss_ref[0, 0] = jnp.mean(log_z - p_y)
        pred = jnp.min(jnp.where(pooled == m, col, D), axis=1, keepdims=True)
        acc_ref[0, 0] = jnp.mean((pred == y).astype(jnp.float32))

    loss, acc = pl.pallas_call(
        head,
        out_shape=[
            jax.ShapeDtypeStruct((1, 1), jnp.float32),
            jax.ShapeDtypeStruct((1, 1), jnp.float32),
        ],
        out_specs=[
            pl.BlockSpec(memory_space=pltpu.SMEM),
            pl.BlockSpec(memory_space=pltpu.SMEM),
        ],
    )(sums, data, t1)
    return loss[0, 0], acc[0, 0]


def kernel(data, table, sent_maxlen):
    B, W = data.shape
    sums = _sc_gather_sum(data.reshape(-1), table, B, W)
    return _tc_head(sums, data, table[1:2, :])


# speedup vs baseline: 15.3624x; 1.2352x over previous
import functools

import jax
import jax.numpy as jnp
from jax import lax
from jax.experimental import pallas as pl
from jax.experimental.pallas import tpu as pltpu
from jax.experimental.pallas import tpu_sc as plsc

_NC = 2
_NS = 16
_L = 16


def _sc_gather_sum(data_flat, table, B, W):
    S = W - 2
    V, D = table.shape
    NW = _NC * _NS
    BPW = B // NW
    G = 4
    GW = G * W
    NG = BPW // G
    NCH = (GW + 127) // 128

    CHUNKS = [(c * 128, min(128, GW - c * 128)) for c in range(NCH)]

    mesh = plsc.VectorSubcoreMesh(core_axis_name="c", subcore_axis_name="s")

    @functools.partial(
        pl.kernel,
        out_type=jax.ShapeDtypeStruct((B, D), jnp.float32),
        mesh=mesh,
        scratch_types=[
            pltpu.VMEM((GW,), jnp.int32),
            pltpu.VMEM((GW,), jnp.int32),
            pltpu.VMEM((GW, D), jnp.float32),
            pltpu.VMEM((GW, D), jnp.float32),
            pltpu.VMEM((BPW, D), jnp.float32),
            pltpu.SemaphoreType.DMA,
            pltpu.SemaphoreType.DMA,
            pltpu.SemaphoreType.DMA,
            pltpu.SemaphoreType.DMA,
        ],
        compiler_params=pltpu.CompilerParams(use_tc_tiling_on_sc=False),
    )
    def sc_kernel(data_hbm, table_hbm, out_hbm, dbuf0, dbuf1, rows0, rows1,
                  outbuf, gsem0, gsem1, dsem0, dsem1):
        wid = lax.axis_index("s") * _NC + lax.axis_index("c")
        base = wid * BPW

        def fetch(g, dbuf, dsem):
            gc = jnp.minimum(g, NG - 1)
            off = (base + gc * G) * W
            return pltpu.async_copy(data_hbm.at[pl.ds(off, GW)], dbuf, dsem)

        def launch_gathers(dbuf, rows, gsem):
            for cs, cl in CHUNKS:
                pltpu.async_copy(table_hbm.at[dbuf.at[pl.ds(cs, cl)]],
                                 rows.at[pl.ds(cs, cl)], gsem)

        def drain_gathers(dbuf, rows, gsem):
            for cs, cl in CHUNKS:
                pltpu.make_async_copy(table_hbm.at[dbuf.at[pl.ds(cs, cl)]],
                                      rows.at[pl.ds(cs, cl)], gsem).wait()

        def drain_fetch(g, dbuf, dsem):
            gc = jnp.minimum(g, NG - 1)
            off = (base + gc * G) * W
            pltpu.make_async_copy(data_hbm.at[pl.ds(off, GW)], dbuf,
                                  dsem).wait()

        def accumulate(g, rows):
            for j in range(G):
                jo = j * W

                def acc_step(k, carry, jo=jo):
                    a0, b0, a1, b1, a2, b2, a3, b3 = carry
                    t = jo + k * 4
                    a0 = a0 + rows[t, pl.ds(0, _L)]
                    b0 = b0 + rows[t, pl.ds(_L, _L)]
                    a1 = a1 + rows[t + 1, pl.ds(0, _L)]
                    b1 = b1 + rows[t + 1, pl.ds(_L, _L)]
                    a2 = a2 + rows[t + 2, pl.ds(0, _L)]
                    b2 = b2 + rows[t + 2, pl.ds(_L, _L)]
                    a3 = a3 + rows[t + 3, pl.ds(0, _L)]
                    b3 = b3 + rows[t + 3, pl.ds(_L, _L)]
                    return a0, b0, a1, b1, a2, b2, a3, b3

                z = jnp.zeros((_L,), jnp.float32)
                a0, b0, a1, b1, a2, b2, a3, b3 = plsc.parallel_loop(
                    0, S // 4, 1, unroll=2,
                    carry=(z, z, z, z, z, z, z, z))(acc_step)
                outbuf[g * G + j, pl.ds(0, _L)] = (a0 + a1) + (a2 + a3)
                outbuf[g * G + j, pl.ds(_L, _L)] = (b0 + b1) + (b2 + b3)

        fetch(0, dbuf0, dsem0).wait()
        launch_gathers(dbuf0, rows0, gsem0)
        fetch(1, dbuf1, dsem1)

        def pair_step(p, _):
            g0 = 2 * p
            drain_gathers(dbuf0, rows0, gsem0)
            fetch(g0 + 2, dbuf0, dsem0)
            drain_fetch(g0 + 1, dbuf1, dsem1)
            launch_gathers(dbuf1, rows1, gsem1)
            accumulate(g0, rows0)

            drain_gathers(dbuf1, rows1, gsem1)
            fetch(g0 + 3, dbuf1, dsem1)
            drain_fetch(g0 + 2, dbuf0, dsem0)
            launch_gathers(dbuf0, rows0, gsem0)
            accumulate(g0 + 1, rows1)
            return 0

        lax.fori_loop(0, NG // 2, pair_step, 0)
        drain_gathers(dbuf0, rows0, gsem0)
        drain_fetch(NG + 1, dbuf1, dsem1)
        pltpu.sync_copy(outbuf, out_hbm.at[pl.ds(base, BPW)])

    return sc_kernel(data_flat, table)


def _tc_head(sums, data, t1):
    B, D = sums.shape
    S = data.shape[1] - 2

    def head(sums_ref, data_ref, t1_ref, loss_ref, acc_ref):
        tokens = data_ref[:, :S]
        cnt = jnp.sum((tokens == 1).astype(jnp.float32), axis=1, keepdims=True)
        lens = data_ref[:, S:S + 1].astype(jnp.float32)
        y = data_ref[:, S + 1:S + 2]
        pooled = (sums_ref[...] - cnt * t1_ref[...]) / lens
        col = lax.broadcasted_iota(jnp.int32, (B, D), 1)
        m = jnp.max(pooled, axis=1, keepdims=True)
        se = jnp.sum(jnp.exp(pooled - m), axis=1, keepdims=True)
        log_z = m + jnp.log(se)
        p_y = jnp.sum(jnp.where(col == y, pooled, 0.0), axis=1, keepdims=True)
        loss_ref[0, 0] = jnp.mean(log_z - p_y)
        pred = jnp.min(jnp.where(pooled == m, col, D), axis=1, keepdims=True)
        acc_ref[0, 0] = jnp.mean((pred == y).astype(jnp.float32))

    loss, acc = pl.pallas_call(
        head,
        out_shape=[
            jax.ShapeDtypeStruct((1, 1), jnp.float32),
            jax.ShapeDtypeStruct((1, 1), jnp.float32),
        ],
        out_specs=[
            pl.BlockSpec(memory_space=pltpu.SMEM),
            pl.BlockSpec(memory_space=pltpu.SMEM),
        ],
    )(sums, data, t1)
    return loss[0, 0], acc[0, 0]


def kernel(data, table, sent_maxlen):
    B, W = data.shape
    sums = _sc_gather_sum(data.reshape(-1), table, B, W)
    return _tc_head(sums, data, table[1:2, :])


# speedup vs baseline: 15.6093x; 1.0161x over previous
import functools

import jax
import jax.numpy as jnp
from jax import lax
from jax.experimental import pallas as pl
from jax.experimental.pallas import tpu as pltpu
from jax.experimental.pallas import tpu_sc as plsc

_NC = 2
_NS = 16
_L = 16


def _sc_gather_sum(data_flat, table, B, W):
    S = W - 2
    V, D = table.shape
    NW = _NC * _NS
    BPW = B // NW
    G = 4
    GW = G * W
    NG = BPW // G
    NCH = (GW + 127) // 128

    CHUNKS = [(c * 128, min(128, GW - c * 128)) for c in range(NCH)]

    mesh = plsc.VectorSubcoreMesh(core_axis_name="c", subcore_axis_name="s")

    @functools.partial(
        pl.kernel,
        out_type=jax.ShapeDtypeStruct((B, D), jnp.float32),
        mesh=mesh,
        scratch_types=[
            pltpu.VMEM((GW,), jnp.int32),
            pltpu.VMEM((GW,), jnp.int32),
            pltpu.VMEM((GW, D), jnp.float32),
            pltpu.VMEM((GW, D), jnp.float32),
            pltpu.VMEM((BPW, D), jnp.float32),
            pltpu.SemaphoreType.DMA,
            pltpu.SemaphoreType.DMA,
            pltpu.SemaphoreType.DMA,
            pltpu.SemaphoreType.DMA,
        ],
        compiler_params=pltpu.CompilerParams(use_tc_tiling_on_sc=False),
    )
    def sc_kernel(data_hbm, table_hbm, out_hbm, dbuf0, dbuf1, rows0, rows1,
                  outbuf, gsem0, gsem1, dsem0, dsem1):
        wid = lax.axis_index("s") * _NC + lax.axis_index("c")
        base = wid * BPW

        def fetch(g, dbuf, dsem):
            gc = jnp.minimum(g, NG - 1)
            off = (base + gc * G) * W
            return pltpu.async_copy(data_hbm.at[pl.ds(off, GW)], dbuf, dsem)

        def launch_gathers(dbuf, rows, gsem):
            for cs, cl in CHUNKS:
                pltpu.async_copy(table_hbm.at[dbuf.at[pl.ds(cs, cl)]],
                                 rows.at[pl.ds(cs, cl)], gsem)

        def drain_gathers(dbuf, rows, gsem):
            for cs, cl in CHUNKS:
                pltpu.make_async_copy(table_hbm.at[dbuf.at[pl.ds(cs, cl)]],
                                      rows.at[pl.ds(cs, cl)], gsem).wait()

        def drain_fetch(g, dbuf, dsem):
            gc = jnp.minimum(g, NG - 1)
            off = (base + gc * G) * W
            pltpu.make_async_copy(data_hbm.at[pl.ds(off, GW)], dbuf,
                                  dsem).wait()

        def accumulate(g, rows):
            for j in range(G):
                jo = j * W

                def acc_step(k, carry, jo=jo):
                    a0, b0, a1, b1, a2, b2, a3, b3 = carry
                    t = jo + k * 4
                    a0 = a0 + rows[t, pl.ds(0, _L)]
                    b0 = b0 + rows[t, pl.ds(_L, _L)]
                    a1 = a1 + rows[t + 1, pl.ds(0, _L)]
                    b1 = b1 + rows[t + 1, pl.ds(_L, _L)]
                    a2 = a2 + rows[t + 2, pl.ds(0, _L)]
                    b2 = b2 + rows[t + 2, pl.ds(_L, _L)]
                    a3 = a3 + rows[t + 3, pl.ds(0, _L)]
                    b3 = b3 + rows[t + 3, pl.ds(_L, _L)]
                    return a0, b0, a1, b1, a2, b2, a3, b3

                z = jnp.zeros((_L,), jnp.float32)
                a0, b0, a1, b1, a2, b2, a3, b3 = plsc.parallel_loop(
                    0, S // 4, 1, unroll=2,
                    carry=(z, z, z, z, z, z, z, z))(acc_step)
                outbuf[g * G + j, pl.ds(0, _L)] = (a0 + a1) + (a2 + a3)
                outbuf[g * G + j, pl.ds(_L, _L)] = (b0 + b1) + (b2 + b3)

        fetch(0, dbuf0, dsem0).wait()
        launch_gathers(dbuf0, rows0, gsem0)
        fetch(1, dbuf1, dsem1)

        def pair_step(p, _):
            g0 = 2 * p
            drain_gathers(dbuf0, rows0, gsem0)
            fetch(g0 + 2, dbuf0, dsem0)
            drain_fetch(g0 + 1, dbuf1, dsem1)
            launch_gathers(dbuf1, rows1, gsem1)
            accumulate(g0, rows0)

            drain_gathers(dbuf1, rows1, gsem1)
            fetch(g0 + 3, dbuf1, dsem1)
            drain_fetch(g0 + 2, dbuf0, dsem0)
            launch_gathers(dbuf0, rows0, gsem0)
            accumulate(g0 + 1, rows1)
            return 0

        lax.fori_loop(0, NG // 2, pair_step, 0)
        drain_gathers(dbuf0, rows0, gsem0)
        drain_fetch(NG + 1, dbuf1, dsem1)
        pltpu.sync_copy(outbuf, out_hbm.at[pl.ds(base, BPW)])

    return sc_kernel(data_flat, table)


def _tc_count(data, S):
    B = data.shape[0]

    def body(data_ref, cnt_ref):
        tokens = data_ref[:, :S]
        cnt_ref[...] = jnp.sum((tokens == 1).astype(jnp.float32), axis=1,
                               keepdims=True)

    return pl.pallas_call(
        body,
        out_shape=jax.ShapeDtypeStruct((B, 1), jnp.float32),
    )(data)


def _tc_head(sums_t, cnt_t, lens_t, labels_t, t1_t):
    D, B = sums_t.shape

    def head(sums_ref, cnt_ref, lens_ref, labels_ref, t1_ref, loss_ref,
             acc_ref):
        y = labels_ref[...]
        pooled = ((sums_ref[...] - t1_ref[...] * cnt_ref[...])
                  / lens_ref[...].astype(jnp.float32))
        row = lax.broadcasted_iota(jnp.int32, (D, B), 0)
        m = jnp.max(pooled, axis=0, keepdims=True)
        se = jnp.sum(jnp.exp(pooled - m), axis=0, keepdims=True)
        log_z = m + jnp.log(se)
        p_y = jnp.sum(jnp.where(row == y, pooled, 0.0), axis=0, keepdims=True)
        loss_ref[0, 0] = jnp.mean(log_z - p_y)
        pred = jnp.min(jnp.where(pooled == m, row, D), axis=0, keepdims=True)
        acc_ref[0, 0] = jnp.mean((pred == y).astype(jnp.float32))

    loss, acc = pl.pallas_call(
        head,
        out_shape=[
            jax.ShapeDtypeStruct((1, 1), jnp.float32),
            jax.ShapeDtypeStruct((1, 1), jnp.float32),
        ],
        out_specs=[
            pl.BlockSpec(memory_space=pltpu.SMEM),
            pl.BlockSpec(memory_space=pltpu.SMEM),
        ],
    )(sums_t, cnt_t, lens_t, labels_t, t1_t)
    return loss[0, 0], acc[0, 0]


def kernel(data, table, sent_maxlen):
    B, W = data.shape
    S = W - 2
    cnt = _tc_count(data, S)
    sums = _sc_gather_sum(data.reshape(-1), table, B, W)
    lens_t = data[:, S:S + 1].T
    labels_t = data[:, S + 1:S + 2].T
    return _tc_head(sums.T, cnt.T, lens_t, labels_t, table[1:2, :].T)
